# baseline (speedup 1.0000x reference)
import functools

import jax
import jax.numpy as jnp
from jax import lax
from jax.experimental import pallas as pl
from jax.experimental.pallas import tpu as pltpu
from jax.experimental.pallas import tpu_sc as plsc

_NC = 2
_NS = 16
_LANES = 16
_NW = _NC * _NS

_IDX_CHUNK = 128


@functools.lru_cache(maxsize=None)
def _make_gather_sum(V, D, B, L):
    assert B % _NW == 0 and D % _LANES == 0
    bw = B // _NW
    rows = bw * L
    assert rows % _IDX_CHUNK == 0
    n_chunks = rows // _IDX_CHUNK
    mesh = plsc.VectorSubcoreMesh(core_axis_name="c", subcore_axis_name="s")

    def body(idx_hbm, emb_hbm, out_hbm, idx_v, rows_v, acc_v, sem):
        wid = lax.axis_index("s") * _NC + lax.axis_index("c")
        pltpu.sync_copy(idx_hbm.at[wid], idx_v)
        copies = []
        for j in range(n_chunks):
            copies.append(
                pltpu.async_copy(
                    emb_hbm.at[idx_v.at[j]],
                    rows_v.at[pl.ds(j * _IDX_CHUNK, _IDX_CHUNK)],
                    sem,
                )
            )
        for c in copies:
            c.wait()

        nd = D // _LANES

        @pl.loop(0, bw)
        def _(i):
            base = i * L
            for d in range(nd):
                acc = rows_v[base, pl.ds(d * _LANES, _LANES)]
                for l in range(1, L):
                    acc = acc + rows_v[base + l, pl.ds(d * _LANES, _LANES)]
                acc_v[i, pl.ds(d * _LANES, _LANES)] = acc

        pltpu.sync_copy(acc_v, out_hbm.at[pl.ds(wid * bw, bw)])

    return pl.kernel(
        body,
        out_type=jax.ShapeDtypeStruct((B, D), jnp.float32),
        mesh=mesh,
        scratch_types=[
            pltpu.VMEM((n_chunks, _IDX_CHUNK), jnp.int32),
            pltpu.VMEM((rows, D), jnp.float32),
            pltpu.VMEM((bw, D), jnp.float32),
            pltpu.SemaphoreType.DMA,
        ],
        compiler_params=pltpu.CompilerParams(use_tc_tiling_on_sc=False),
    )


@functools.lru_cache(maxsize=None)
def _make_mlp(B, D, H, V, L, VB, NBUF):
    inv_l = 1.0 / L
    NJ = pl.cdiv(V, VB)
    TV = V - (NJ - 1) * VB
    assert NJ > NBUF and TV % 8 == 0

    def body(sums_ref, whT_ref, bhT_ref, wfcT_ref, bfc_ref, outT_hbm, obuf, hT_buf, sems):
        j = pl.program_id(0)
        slot = lax.rem(j, NBUF)

        @pl.when(j >= NBUF)
        def _():
            jm = j - NBUF
            pltpu.make_async_copy(
                obuf.at[slot],
                outT_hbm.at[pl.ds(jm * VB, VB), :],
                sems.at[slot],
            ).wait()

        @pl.when(j == 0)
        def _():
            avgT = jnp.transpose(sums_ref[...]) * inv_l
            t = jnp.dot(whT_ref[...], avgT, preferred_element_type=jnp.float32)
            hT_buf[...] = jnp.maximum(t + bhT_ref[...], 0.0)

        hT = hT_buf[...]
        bcol = jnp.transpose(bfc_ref[...])
        obuf[slot] = (
            jnp.dot(wfcT_ref[...], hT, preferred_element_type=jnp.float32)
            + bcol
        )

        @pl.when(j < NJ - 1)
        def _():
            pltpu.async_copy(
                obuf.at[slot], outT_hbm.at[pl.ds(j * VB, VB), :], sems.at[slot]
            )

        @pl.when(j == NJ - 1)
        def _():
            pltpu.async_copy(
                obuf.at[slot, pl.ds(0, TV), :],
                outT_hbm.at[pl.ds(j * VB, TV), :],
                sems.at[slot],
            )
            for k in range(NBUF):
                jm = NJ - NBUF + k
                s = jm % NBUF
                w = VB if jm < NJ - 1 else TV
                pltpu.make_async_copy(
                    obuf.at[s, pl.ds(0, w), :],
                    outT_hbm.at[pl.ds(jm * VB, w), :],
                    sems.at[s],
                ).wait()

    return pl.pallas_call(
        body,
        grid=(NJ,),
        in_specs=[
            pl.BlockSpec((B, D), lambda j: (0, 0)),
            pl.BlockSpec((H, D), lambda j: (0, 0)),
            pl.BlockSpec((H, 1), lambda j: (0, 0)),
            pl.BlockSpec((VB, D), lambda j: (j, 0)),
            pl.BlockSpec((1, VB), lambda j: (0, j)),
        ],
        out_specs=pl.BlockSpec(memory_space=pltpu.HBM),
        out_shape=jax.ShapeDtypeStruct((V, B), jnp.float32),
        scratch_shapes=[
            pltpu.VMEM((NBUF, VB, B), jnp.float32),
            pltpu.VMEM((H, B), jnp.float32),
            pltpu.SemaphoreType.DMA((NBUF,)),
        ],
        compiler_params=pltpu.CompilerParams(
            dimension_semantics=("arbitrary",),
        ),
    )


def kernel(input, emb, W_h, b_h, W_fc, b_fc):
    B, L = input.shape
    V, D = emb.shape
    H = W_h.shape[1]
    rows = (B // _NW) * L
    idx = input.astype(jnp.int32).reshape(_NW, rows // _IDX_CHUNK, _IDX_CHUNK)
    sums = _make_gather_sum(V, D, B, L)(idx, emb)
    outT = _make_mlp(B, D, H, V, L, 2048, 4)(
        sums, W_h.T, b_h.reshape(H, 1), W_fc.T, b_fc.reshape(1, V)
    )
    return outT.T


# speedup vs baseline: 1.0576x; 1.0576x over previous
import functools

import jax
import jax.numpy as jnp
from jax import lax
from jax.experimental import pallas as pl
from jax.experimental.pallas import tpu as pltpu
from jax.experimental.pallas import tpu_sc as plsc

_NC = 2
_NS = 16
_LANES = 16
_NW = _NC * _NS

_IDX_CHUNK = 128


@functools.lru_cache(maxsize=None)
def _make_gather_sum(V, D, B, L):
    assert B % _NW == 0 and D % _LANES == 0 and 2 * D == 128
    bw = B // _NW
    hw = bw // 2
    rows = bw * L
    assert rows % _IDX_CHUNK == 0
    n_chunks = rows // _IDX_CHUNK
    mesh = plsc.VectorSubcoreMesh(core_axis_name="c", subcore_axis_name="s")

    def body(idx_hbm, emb_hbm, out_hbm, idx_v, rows_v, acc_v, sem):
        wid = lax.axis_index("s") * _NC + lax.axis_index("c")
        pltpu.sync_copy(idx_hbm.at[wid], idx_v)
        copies = []
        for j in range(n_chunks):
            copies.append(
                pltpu.async_copy(
                    emb_hbm.at[idx_v.at[j]],
                    rows_v.at[pl.ds(j * _IDX_CHUNK, _IDX_CHUNK)],
                    sem,
                )
            )
        for c in copies:
            c.wait()

        nd = D // _LANES
        for half in range(2):
            @pl.loop(0, hw)
            def _(i, half=half):
                base = (half * hw + i) * L
                for d in range(nd):
                    acc = rows_v[base, pl.ds(d * _LANES, _LANES)]
                    for l in range(1, L):
                        acc = acc + rows_v[base + l, pl.ds(d * _LANES, _LANES)]
                    acc_v[i, pl.ds(half * D + d * _LANES, _LANES)] = acc

        pltpu.sync_copy(acc_v, out_hbm.at[pl.ds(wid * hw, hw)])

    return pl.kernel(
        body,
        out_type=jax.ShapeDtypeStruct((B // 2, 2 * D), jnp.float32),
        mesh=mesh,
        scratch_types=[
            pltpu.VMEM((n_chunks, _IDX_CHUNK), jnp.int32),
            pltpu.VMEM((rows, 2 * D), jnp.float32),
            pltpu.VMEM((hw, 2 * D), jnp.float32),
            pltpu.SemaphoreType.DMA,
        ],
        compiler_params=pltpu.CompilerParams(use_tc_tiling_on_sc=False),
    )


@functools.lru_cache(maxsize=None)
def _make_mlp(B, D, H, V, L, VB, NBUF):
    inv_l = 1.0 / L
    NJ = pl.cdiv(V, VB)
    TV = V - (NJ - 1) * VB
    assert NJ > NBUF and TV % 8 == 0

    def body(sums_ref, whT_ref, bhT_ref, wfcT_ref, bfc_ref, outT_hbm, obuf, hT_buf, sems):
        j = pl.program_id(0)
        slot = lax.rem(j, NBUF)

        @pl.when(j >= NBUF)
        def _():
            jm = j - NBUF
            pltpu.make_async_copy(
                obuf.at[slot],
                outT_hbm.at[pl.ds(jm * VB, VB), :],
                sems.at[slot],
            ).wait()

        @pl.when(j == 0)
        def _():
            s = sums_ref[...]
            cat = jnp.concatenate([s[:, :D], s[:, D:]], axis=0)
            avgT = jnp.transpose(cat) * inv_l
            t = jnp.dot(whT_ref[...], avgT, preferred_element_type=jnp.float32)
            hT_buf[...] = jnp.maximum(t + bhT_ref[...], 0.0)

        hT = hT_buf[...]
        bcol = jnp.transpose(bfc_ref[...])
        obuf[slot] = (
            jnp.dot(wfcT_ref[...], hT, preferred_element_type=jnp.float32)
            + bcol
        )

        @pl.when(j < NJ - 1)
        def _():
            pltpu.async_copy(
                obuf.at[slot], outT_hbm.at[pl.ds(j * VB, VB), :], sems.at[slot]
            )

        @pl.when(j == NJ - 1)
        def _():
            pltpu.async_copy(
                obuf.at[slot, pl.ds(0, TV), :],
                outT_hbm.at[pl.ds(j * VB, TV), :],
                sems.at[slot],
            )
            for k in range(NBUF):
                jm = NJ - NBUF + k
                s = jm % NBUF
                w = VB if jm < NJ - 1 else TV
                pltpu.make_async_copy(
                    obuf.at[s, pl.ds(0, w), :],
                    outT_hbm.at[pl.ds(jm * VB, w), :],
                    sems.at[s],
                ).wait()

    return pl.pallas_call(
        body,
        grid=(NJ,),
        in_specs=[
            pl.BlockSpec((B // 2, 2 * D), lambda j: (0, 0)),
            pl.BlockSpec((H, D), lambda j: (0, 0)),
            pl.BlockSpec((H, 1), lambda j: (0, 0)),
            pl.BlockSpec((VB, D), lambda j: (j, 0)),
            pl.BlockSpec((1, VB), lambda j: (0, j)),
        ],
        out_specs=pl.BlockSpec(memory_space=pltpu.HBM),
        out_shape=jax.ShapeDtypeStruct((V, B), jnp.float32),
        scratch_shapes=[
            pltpu.VMEM((NBUF, VB, B), jnp.float32),
            pltpu.VMEM((H, B), jnp.float32),
            pltpu.SemaphoreType.DMA((NBUF,)),
        ],
        compiler_params=pltpu.CompilerParams(
            dimension_semantics=("arbitrary",),
        ),
    )


def kernel(input, emb, W_h, b_h, W_fc, b_fc):
    B, L = input.shape
    V, D = emb.shape
    H = W_h.shape[1]
    rows = (B // _NW) * L
    hw_l = (B // _NW // 2) * L
    inp32 = input.astype(jnp.int32)
    idx = jnp.concatenate(
        [inp32[: B // 2].reshape(_NW, hw_l), inp32[B // 2 :].reshape(_NW, hw_l)],
        axis=1,
    ).reshape(_NW, rows // _IDX_CHUNK, _IDX_CHUNK)
    emb128 = jnp.pad(emb, ((0, 0), (0, 128 - D)))
    sums = _make_gather_sum(V, D, B, L)(idx, emb128)
    outT = _make_mlp(B, D, H, V, L, 2048, 4)(
        sums, W_h.T, b_h.reshape(H, 1), W_fc.T, b_fc.reshape(1, V)
    )
    return outT.T


# speedup vs baseline: 1.0577x; 1.0001x over previous
import functools

import jax
import jax.numpy as jnp
from jax import lax
from jax.experimental import pallas as pl
from jax.experimental.pallas import tpu as pltpu
from jax.experimental.pallas import tpu_sc as plsc

_NC = 2
_NS = 16
_LANES = 16
_NW = _NC * _NS

_IDX_CHUNK = 128


@functools.lru_cache(maxsize=None)
def _make_gather_sum(V, D, B, L):
    assert B % _NW == 0 and D % _LANES == 0 and 2 * D == 128
    bw = B // _NW
    hw = bw // 2
    rows = bw * L
    assert rows % _IDX_CHUNK == 0
    n_chunks = rows // _IDX_CHUNK
    mesh = plsc.VectorSubcoreMesh(core_axis_name="c", subcore_axis_name="s")

    def body(idx_hbm, emb_hbm, out_hbm, idx_v, rows_v, acc_v, sem):
        wid = lax.axis_index("s") * _NC + lax.axis_index("c")
        pltpu.sync_copy(idx_hbm.at[wid], idx_v)
        copies = []
        for j in range(n_chunks):
            copies.append(
                pltpu.async_copy(
                    emb_hbm.at[idx_v.at[j]],
                    rows_v.at[pl.ds(j * _IDX_CHUNK, _IDX_CHUNK)],
                    sem,
                )
            )
        for c in copies:
            c.wait()

        nd = D // _LANES
        for half in range(2):
            @pl.loop(0, hw)
            def _(i, half=half):
                base = (half * hw + i) * L
                for d in range(nd):
                    acc = rows_v[base, pl.ds(d * _LANES, _LANES)]
                    for l in range(1, L):
                        acc = acc + rows_v[base + l, pl.ds(d * _LANES, _LANES)]
                    acc_v[i, pl.ds(half * D + d * _LANES, _LANES)] = acc

        pltpu.sync_copy(acc_v, out_hbm.at[pl.ds(wid * hw, hw)])

    return pl.kernel(
        body,
        out_type=jax.ShapeDtypeStruct((B // 2, 2 * D), jnp.float32),
        mesh=mesh,
        scratch_types=[
            pltpu.VMEM((n_chunks, _IDX_CHUNK), jnp.int32),
            pltpu.VMEM((rows, D), jnp.float32),
            pltpu.VMEM((hw, 2 * D), jnp.float32),
            pltpu.SemaphoreType.DMA,
        ],
        compiler_params=pltpu.CompilerParams(use_tc_tiling_on_sc=False),
    )


@functools.lru_cache(maxsize=None)
def _make_mlp(B, D, H, V, L, VB, NBUF):
    inv_l = 1.0 / L
    NJ = pl.cdiv(V, VB)
    TV = V - (NJ - 1) * VB
    assert NJ > NBUF and TV % 8 == 0

    def body(sums_ref, whT_ref, bhT_ref, wfcT_ref, bfc_ref, outT_hbm, obuf, hT_buf, sems):
        j = pl.program_id(0)
        slot = lax.rem(j, NBUF)

        @pl.when(j >= NBUF)
        def _():
            jm = j - NBUF
            pltpu.make_async_copy(
                obuf.at[slot],
                outT_hbm.at[pl.ds(jm * VB, VB), :],
                sems.at[slot],
            ).wait()

        @pl.when(j == 0)
        def _():
            s = sums_ref[...]
            cat = jnp.concatenate([s[:, :D], s[:, D:]], axis=0)
            avgT = jnp.transpose(cat) * inv_l
            t = jnp.dot(whT_ref[...], avgT, preferred_element_type=jnp.float32)
            hT_buf[...] = jnp.maximum(t + bhT_ref[...], 0.0)

        hT = hT_buf[...]
        bcol = jnp.transpose(bfc_ref[...])
        obuf[slot] = (
            jnp.dot(wfcT_ref[...], hT, preferred_element_type=jnp.float32)
            + bcol
        )

        @pl.when(j < NJ - 1)
        def _():
            pltpu.async_copy(
                obuf.at[slot], outT_hbm.at[pl.ds(j * VB, VB), :], sems.at[slot]
            )

        @pl.when(j == NJ - 1)
        def _():
            pltpu.async_copy(
                obuf.at[slot, pl.ds(0, TV), :],
                outT_hbm.at[pl.ds(j * VB, TV), :],
                sems.at[slot],
            )
            for k in range(NBUF):
                jm = NJ - NBUF + k
                s = jm % NBUF
                w = VB if jm < NJ - 1 else TV
                pltpu.make_async_copy(
                    obuf.at[s, pl.ds(0, w), :],
                    outT_hbm.at[pl.ds(jm * VB, w), :],
                    sems.at[s],
                ).wait()

    return pl.pallas_call(
        body,
        grid=(NJ,),
        in_specs=[
            pl.BlockSpec((B // 2, 2 * D), lambda j: (0, 0)),
            pl.BlockSpec((H, D), lambda j: (0, 0)),
            pl.BlockSpec((H, 1), lambda j: (0, 0)),
            pl.BlockSpec((VB, D), lambda j: (j, 0)),
            pl.BlockSpec((1, VB), lambda j: (0, j)),
        ],
        out_specs=pl.BlockSpec(memory_space=pltpu.HBM),
        out_shape=jax.ShapeDtypeStruct((V, B), jnp.float32),
        scratch_shapes=[
            pltpu.VMEM((NBUF, VB, B), jnp.float32),
            pltpu.VMEM((H, B), jnp.float32),
            pltpu.SemaphoreType.DMA((NBUF,)),
        ],
        compiler_params=pltpu.CompilerParams(
            dimension_semantics=("arbitrary",),
        ),
    )


def kernel(input, emb, W_h, b_h, W_fc, b_fc):
    B, L = input.shape
    V, D = emb.shape
    H = W_h.shape[1]
    rows = (B // _NW) * L
    hw_l = (B // _NW // 2) * L
    inp32 = input.astype(jnp.int32)
    idx = jnp.concatenate(
        [inp32[: B // 2].reshape(_NW, hw_l), inp32[B // 2 :].reshape(_NW, hw_l)],
        axis=1,
    ).reshape(_NW, rows // _IDX_CHUNK, _IDX_CHUNK)
    sums = _make_gather_sum(V, D, B, L)(idx, emb)
    outT = _make_mlp(B, D, H, V, L, 2048, 4)(
        sums, W_h.T, b_h.reshape(H, 1), W_fc.T, b_fc.reshape(1, V)
    )
    return outT.T
